# baseline (speedup 1.0000x reference)
import functools

import jax
import jax.numpy as jnp
from jax.experimental import pallas as pl
from jax.experimental.pallas import tpu as pltpu
from jax.experimental.pallas import tpu_sc as plsc

_N = 10000
_E = 320000
_NUM_GRAPHS = 100
_NUM_CLASSES = 6

_NC = 2
_NS = 16
_NTILES = _NC * _NS
_K = 128
_TOT_CHUNKS = _E // _K
_CHUNKS_BASE = _TOT_CHUNKS // _NTILES
_EXTRA_TILES = _TOT_CHUNKS % _NTILES
_ACC_N = 10240
_ROWS_PER_SUB = _ACC_N // _NS

_BLK = 10000
_NBLK = _N // _BLK


def _elu(v):
    return jnp.where(v > 0, v, jnp.exp(v) - 1.0)


@functools.lru_cache(maxsize=None)
def _make_sc_aggregate(D):
    mesh = plsc.VectorSubcoreMesh(core_axis_name="c", subcore_axis_name="s")
    n_halves = 1 if D == 64 else 2
    half_n = _CHUNKS_BASE // n_halves

    @functools.partial(
        pl.kernel,
        out_type=jax.ShapeDtypeStruct((_NC, _ACC_N, D), jnp.float32),
        mesh=mesh,
        compiler_params=pltpu.CompilerParams(use_tc_tiling_on_sc=False),
        scratch_types=[
            pltpu.VMEM((half_n, _K), jnp.int32),
            pltpu.VMEM((half_n, _K), jnp.int32),
            pltpu.VMEM((1, _K), jnp.int32),
            pltpu.VMEM((1, _K), jnp.int32),
            pltpu.VMEM((_K, D), jnp.float32),
            pltpu.VMEM((_K, D), jnp.float32),
            pltpu.VMEM_SHARED((_ACC_N, D), jnp.float32),
            pltpu.SemaphoreType.DMA,
            pltpu.SemaphoreType.DMA,
            pltpu.SemaphoreType.DMA,
            pltpu.SemaphoreType.DMA,
        ],
    )
    def agg_kernel(h_hbm, src_hbm, dst_hbm, zero_hbm, out_hbm,
                   src_v, dst_v, src_x, dst_x, rows_a, rows_b, acc_sh,
                   gsa, gsb, ssa, ssb):
        rows = (rows_a, rows_b)
        gsem = (gsa, gsb)
        ssem = (ssa, ssb)
        c = jax.lax.axis_index("c")
        s = jax.lax.axis_index("s")
        wid = c * _NS + s
        start_chunk = wid * _CHUNKS_BASE + jnp.minimum(wid, _EXTRA_TILES)
        has_extra = wid < _EXTRA_TILES

        def preload(half):
            pltpu.async_copy(
                src_hbm.at[pl.ds(start_chunk + half * half_n, half_n)],
                src_v, gsa)
            pltpu.async_copy(
                dst_hbm.at[pl.ds(start_chunk + half * half_n, half_n)],
                dst_v, gsb)

        def preload_wait(half):
            pltpu.make_async_copy(
                src_hbm.at[pl.ds(start_chunk + half * half_n, half_n)],
                src_v, gsa).wait()
            pltpu.make_async_copy(
                dst_hbm.at[pl.ds(start_chunk + half * half_n, half_n)],
                dst_v, gsb).wait()

        def g_start(ci, j):
            pltpu.async_copy(h_hbm.at[src_v.at[ci]], rows[j], gsem[j])

        def g_wait(ci, j):
            pltpu.make_async_copy(h_hbm.at[src_v.at[ci]], rows[j],
                                  gsem[j]).wait()

        def s_start(ci, j):
            pltpu.async_copy(rows[j], acc_sh.at[dst_v.at[ci]], ssem[j],
                             add=True)

        def s_wait(ci, j):
            pltpu.make_async_copy(rows[j], acc_sh.at[dst_v.at[ci]],
                                  ssem[j]).wait()

        def step(ci, j, lookahead):
            g_wait(ci, j)
            s_start(ci, j)
            s_wait(ci, j)
            if lookahead is not None:
                g_start(lookahead, j)

        def process(n):
            g_start(0, 0)
            g_start(1, 1)
            n_full_pairs = (n - 2) // 2 if n % 2 == 0 else (n - 3) // 2

            @pl.loop(0, n_full_pairs)
            def _(p):
                step(2 * p, 0, 2 * p + 2)
                step(2 * p + 1, 1, 2 * p + 3)

            if n % 2 == 0:
                step(n - 2, 0, None)
                step(n - 1, 1, None)
            else:
                step(n - 3, 0, n - 1)
                step(n - 2, 1, None)
                step(n - 1, 0, None)

        preload(0)
        pltpu.sync_copy(zero_hbm,
                        acc_sh.at[pl.ds(s * _ROWS_PER_SUB, _ROWS_PER_SUB)])
        preload_wait(0)
        plsc.subcore_barrier()

        process(half_n)
        for half in range(1, n_halves):
            preload(half)
            preload_wait(half)
            process(half_n)

        @pl.when(has_extra)
        def _():
            xc = start_chunk + _CHUNKS_BASE
            pltpu.sync_copy(src_hbm.at[pl.ds(xc, 1)], src_x)
            pltpu.sync_copy(dst_hbm.at[pl.ds(xc, 1)], dst_x)
            pltpu.async_copy(h_hbm.at[src_x.at[0]], rows_a, gsa).wait()
            pltpu.async_copy(rows_a, acc_sh.at[dst_x.at[0]], ssa,
                             add=True).wait()

        plsc.subcore_barrier()
        pltpu.sync_copy(acc_sh.at[pl.ds(s * _ROWS_PER_SUB, _ROWS_PER_SUB)],
                        out_hbm.at[c, pl.ds(s * _ROWS_PER_SUB, _ROWS_PER_SUB)])

    return agg_kernel


def _sc_segment_partials(h, src_p, dst_p, zero_rows):
    return _make_sc_aggregate(h.shape[1])(h, src_p, dst_p, zero_rows)


def _row_spec(d):
    return pl.BlockSpec((_BLK, d), lambda i: (i, 0))


def _agg_spec(core, d):
    return pl.BlockSpec((1, _BLK, d), lambda i: (core, i, 0))


def _full_spec(shape):
    return pl.BlockSpec(shape, lambda i: tuple(0 for _ in shape))


def _edges_body(e_ref, s_ref, d_ref):
    s_ref[...] = e_ref[0].reshape(_TOT_CHUNKS, _K)
    d_ref[...] = e_ref[1].reshape(_TOT_CHUNKS, _K)


def _edges(edge_index):
    return pl.pallas_call(
        _edges_body,
        out_shape=[jax.ShapeDtypeStruct((_TOT_CHUNKS, _K), jnp.int32)] * 2,
    )(edge_index)


def _tc1_body(x_ref, w_ref, xr_ref, xo_ref):
    cat = jnp.dot(x_ref[...], w_ref[...], preferred_element_type=jnp.float32)
    xr_ref[...] = cat[:, :64]
    xo_ref[...] = cat[:, 64:]


def _tc1(x, wcat):
    return pl.pallas_call(
        _tc1_body,
        grid=(_NBLK,),
        in_specs=[_row_spec(128), _full_spec((128, 128))],
        out_specs=[_row_spec(64), _row_spec(64)],
        out_shape=[jax.ShapeDtypeStruct((_N, 64), jnp.float32),
                   jax.ShapeDtypeStruct((_N, 64), jnp.float32)],
    )(x, wcat)


def _tc2_body(agg0_ref, agg1_ref, b1_ref, xo1_ref, w2_ref, h1_ref, xo2_ref):
    h1 = _elu(agg0_ref[0] + agg1_ref[0] + b1_ref[...] + xo1_ref[...])
    h1_ref[...] = h1
    xo2_ref[...] = jnp.dot(h1, w2_ref[...], preferred_element_type=jnp.float32)


def _tc2(agg, b1, xo1, wroot2t):
    return pl.pallas_call(
        _tc2_body,
        grid=(_NBLK,),
        in_specs=[_agg_spec(0, 64), _agg_spec(1, 64), _full_spec((1, 64)),
                  _row_spec(64), _full_spec((64, 128))],
        out_specs=[_row_spec(64), _row_spec(128)],
        out_shape=[jax.ShapeDtypeStruct((_N, 64), jnp.float32),
                   jax.ShapeDtypeStruct((_N, 128), jnp.float32)],
    )(agg, agg, b1, xo1, wroot2t)


def _tc3_body(agg0_ref, agg1_ref, b2_ref, xo2_ref, wrel2_ref, wroot3_ref,
              h2_ref, xo3_ref):
    a = agg0_ref[0] + agg1_ref[0]
    h2 = _elu(jnp.dot(a, wrel2_ref[...], preferred_element_type=jnp.float32)
              + b2_ref[...] + xo2_ref[...])
    h2_ref[...] = h2
    xo3_ref[...] = jnp.dot(h2, wroot3_ref[...],
                           preferred_element_type=jnp.float32)


def _tc3(agg, b2, xo2, wrel2t, wroot3t):
    return pl.pallas_call(
        _tc3_body,
        grid=(_NBLK,),
        in_specs=[_agg_spec(0, 64), _agg_spec(1, 64), _full_spec((1, 128)),
                  _row_spec(128), _full_spec((64, 128)),
                  _full_spec((128, 256))],
        out_specs=[_row_spec(128), _row_spec(256)],
        out_shape=[jax.ShapeDtypeStruct((_N, 128), jnp.float32),
                   jax.ShapeDtypeStruct((_N, 256), jnp.float32)],
    )(agg, agg, b2, xo2, wrel2t, wroot3t)


def _tc4_body(agg0_ref, agg1_ref, b3_ref, xo3_ref, wrel3_ref, batch_ref,
              wfc1_ref, bfc1_ref, wfc3_ref, bfc3_ref, o_ref,
              pooled_ref, cnt_ref):
    a = agg0_ref[0] + agg1_ref[0]
    h3 = _elu(jnp.dot(a, wrel3_ref[...], preferred_element_type=jnp.float32)
              + b3_ref[...] + xo3_ref[...])
    b = batch_ref[0, 0, :]
    gids = jax.lax.broadcasted_iota(jnp.int32, (128, _BLK), 0)
    maskt = (gids == b[None, :]).astype(jnp.float32)
    pooled_blk = jnp.dot(maskt, h3, preferred_element_type=jnp.float32)
    cnt_blk = jnp.sum(maskt, axis=1, keepdims=True)

    @pl.when(pl.program_id(0) == 0)
    def _():
        pooled_ref[...] = jnp.zeros_like(pooled_ref)
        cnt_ref[...] = jnp.zeros_like(cnt_ref)

    pooled_ref[...] += pooled_blk
    cnt_ref[...] += cnt_blk

    @pl.when(pl.program_id(0) == _NBLK - 1)
    def _():
        cnt = jnp.maximum(cnt_ref[...], 1.0)
        mean = pooled_ref[...] / cnt
        z = _elu(jnp.dot(mean, wfc1_ref[...],
                         preferred_element_type=jnp.float32)
                 + bfc1_ref[...])
        z2 = (jnp.dot(z, wfc3_ref[...], preferred_element_type=jnp.float32)
              + bfc3_ref[...])
        m = jnp.max(z2, axis=1, keepdims=True)
        ssum = jnp.sum(jnp.exp(z2 - m), axis=1, keepdims=True)
        o_ref[...] = z2 - m - jnp.log(ssum)


def _tc4(agg, b3, xo3, wrel3t, batch3d, wfc1t, bfc1, wfc3t_pad, bfc3_pad):
    return pl.pallas_call(
        _tc4_body,
        grid=(_NBLK,),
        in_specs=[_agg_spec(0, 128), _agg_spec(1, 128), _full_spec((1, 256)),
                  _row_spec(256), _full_spec((128, 256)),
                  pl.BlockSpec((1, 1, _BLK), lambda i: (i, 0, 0)),
                  _full_spec((256, 128)), _full_spec((1, 128)),
                  _full_spec((128, 128)), _full_spec((1, 128))],
        out_specs=_full_spec((128, 128)),
        out_shape=jax.ShapeDtypeStruct((128, 128), jnp.float32),
        scratch_shapes=[pltpu.VMEM((128, 256), jnp.float32),
                        pltpu.VMEM((128, 1), jnp.float32)],
    )(agg, agg, b3, xo3, wrel3t, batch3d, wfc1t, bfc1, wfc3t_pad, bfc3_pad)


def kernel(x, edge_index, batch, W_rel1, b_rel1, W_root1, W_rel2, b_rel2,
           W_root2, W_rel3, b_rel3, W_root3, W_fc1, b_fc1, W_fc3, b_fc3):
    src_p, dst_p = _edges(edge_index.astype(jnp.int32))

    zeros64 = jnp.zeros((_ROWS_PER_SUB, 64), jnp.float32)
    zeros128 = jnp.zeros((_ROWS_PER_SUB, 128), jnp.float32)

    wcat1 = jnp.concatenate([W_rel1.T, W_root1.T], axis=1)
    b1 = b_rel1.reshape(1, 64)
    b2 = b_rel2.reshape(1, 128)
    b3 = b_rel3.reshape(1, 256)
    batch3d = batch.astype(jnp.int32).reshape(_NBLK, 1, _BLK)

    wfc3t_pad = jnp.zeros((128, 128), jnp.float32).at[:, :_NUM_CLASSES].set(
        W_fc3.T)
    bfc3_pad = jnp.full((1, 128), -1e9, jnp.float32).at[0, :_NUM_CLASSES].set(
        b_fc3)

    xr1, xo1 = _tc1(x, wcat1)
    agg1 = _sc_segment_partials(xr1, src_p, dst_p, zeros64)
    h1, xo2 = _tc2(agg1, b1, xo1, W_root2.T)

    agg2 = _sc_segment_partials(h1, src_p, dst_p, zeros64)
    h2, xo3 = _tc3(agg2, b2, xo2, W_rel2.T, W_root3.T)

    agg3 = _sc_segment_partials(h2, src_p, dst_p, zeros128)
    out = _tc4(agg3, b3, xo3, W_rel3.T, batch3d, W_fc1.T,
               b_fc1.reshape(1, 128), wfc3t_pad, bfc3_pad)
    return out[:_NUM_GRAPHS, :_NUM_CLASSES]


# speedup vs baseline: 1.0175x; 1.0175x over previous
import functools

import jax
import jax.numpy as jnp
from jax.experimental import pallas as pl
from jax.experimental.pallas import tpu as pltpu
from jax.experimental.pallas import tpu_sc as plsc

_N = 10000
_E = 320000
_NUM_GRAPHS = 100
_NUM_CLASSES = 6

_NC = 2
_NS = 16
_NTILES = _NC * _NS
_K = 128
_TOT_CHUNKS = _E // _K
_CHUNKS_BASE = _TOT_CHUNKS // _NTILES
_EXTRA_TILES = _TOT_CHUNKS % _NTILES
_ACC_N = 10240
_ROWS_PER_SUB = _ACC_N // _NS

_BLK = 5000
_NBLK = _N // _BLK


def _elu(v):
    return jnp.where(v > 0, v, jnp.exp(v) - 1.0)


@functools.lru_cache(maxsize=None)
def _make_sc_aggregate(D):
    mesh = plsc.VectorSubcoreMesh(core_axis_name="c", subcore_axis_name="s")
    n_halves = 1 if D == 64 else 2
    half_n = _CHUNKS_BASE // n_halves

    @functools.partial(
        pl.kernel,
        out_type=jax.ShapeDtypeStruct((_NC, _ACC_N, D), jnp.float32),
        mesh=mesh,
        compiler_params=pltpu.CompilerParams(use_tc_tiling_on_sc=False),
        scratch_types=[
            pltpu.VMEM((half_n, _K), jnp.int32),
            pltpu.VMEM((half_n, _K), jnp.int32),
            pltpu.VMEM((1, _K), jnp.int32),
            pltpu.VMEM((1, _K), jnp.int32),
            pltpu.VMEM((_K, D), jnp.float32),
            pltpu.VMEM((_K, D), jnp.float32),
            pltpu.VMEM_SHARED((_ACC_N, D), jnp.float32),
            pltpu.SemaphoreType.DMA,
            pltpu.SemaphoreType.DMA,
            pltpu.SemaphoreType.DMA,
            pltpu.SemaphoreType.DMA,
        ],
    )
    def agg_kernel(h_hbm, src_hbm, dst_hbm, zero_hbm, out_hbm,
                   src_v, dst_v, src_x, dst_x, rows_a, rows_b, acc_sh,
                   gsa, gsb, ssa, ssb):
        rows = (rows_a, rows_b)
        gsem = (gsa, gsb)
        ssem = (ssa, ssb)
        c = jax.lax.axis_index("c")
        s = jax.lax.axis_index("s")
        wid = c * _NS + s
        start_chunk = wid * _CHUNKS_BASE + jnp.minimum(wid, _EXTRA_TILES)
        has_extra = wid < _EXTRA_TILES

        def preload(half):
            pltpu.async_copy(
                src_hbm.at[pl.ds(start_chunk + half * half_n, half_n)],
                src_v, gsa)
            pltpu.async_copy(
                dst_hbm.at[pl.ds(start_chunk + half * half_n, half_n)],
                dst_v, gsb)

        def preload_wait(half):
            pltpu.make_async_copy(
                src_hbm.at[pl.ds(start_chunk + half * half_n, half_n)],
                src_v, gsa).wait()
            pltpu.make_async_copy(
                dst_hbm.at[pl.ds(start_chunk + half * half_n, half_n)],
                dst_v, gsb).wait()

        def g_start(ci, j):
            pltpu.async_copy(h_hbm.at[src_v.at[ci]], rows[j], gsem[j])

        def g_wait(ci, j):
            pltpu.make_async_copy(h_hbm.at[src_v.at[ci]], rows[j],
                                  gsem[j]).wait()

        def s_start(ci, j):
            pltpu.async_copy(rows[j], acc_sh.at[dst_v.at[ci]], ssem[j],
                             add=True)

        def s_wait(ci, j):
            pltpu.make_async_copy(rows[j], acc_sh.at[dst_v.at[ci]],
                                  ssem[j]).wait()

        def step(ci, j, lookahead):
            g_wait(ci, j)
            s_start(ci, j)
            s_wait(ci, j)
            if lookahead is not None:
                g_start(lookahead, j)

        def process(n):
            g_start(0, 0)
            g_start(1, 1)
            n_full_pairs = (n - 2) // 2 if n % 2 == 0 else (n - 3) // 2

            @pl.loop(0, n_full_pairs)
            def _(p):
                step(2 * p, 0, 2 * p + 2)
                step(2 * p + 1, 1, 2 * p + 3)

            if n % 2 == 0:
                step(n - 2, 0, None)
                step(n - 1, 1, None)
            else:
                step(n - 3, 0, n - 1)
                step(n - 2, 1, None)
                step(n - 1, 0, None)

        preload(0)
        pltpu.sync_copy(zero_hbm,
                        acc_sh.at[pl.ds(s * _ROWS_PER_SUB, _ROWS_PER_SUB)])
        preload_wait(0)
        plsc.subcore_barrier()

        process(half_n)
        for half in range(1, n_halves):
            preload(half)
            preload_wait(half)
            process(half_n)

        @pl.when(has_extra)
        def _():
            xc = start_chunk + _CHUNKS_BASE
            pltpu.sync_copy(src_hbm.at[pl.ds(xc, 1)], src_x)
            pltpu.sync_copy(dst_hbm.at[pl.ds(xc, 1)], dst_x)
            pltpu.async_copy(h_hbm.at[src_x.at[0]], rows_a, gsa).wait()
            pltpu.async_copy(rows_a, acc_sh.at[dst_x.at[0]], ssa,
                             add=True).wait()

        plsc.subcore_barrier()
        pltpu.sync_copy(acc_sh.at[pl.ds(s * _ROWS_PER_SUB, _ROWS_PER_SUB)],
                        out_hbm.at[c, pl.ds(s * _ROWS_PER_SUB, _ROWS_PER_SUB)])

    return agg_kernel


def _sc_segment_partials(h, src_p, dst_p, zero_rows):
    return _make_sc_aggregate(h.shape[1])(h, src_p, dst_p, zero_rows)


def _row_spec(d):
    return pl.BlockSpec((_BLK, d), lambda i: (i, 0))


def _agg_spec(core, d):
    return pl.BlockSpec((1, _BLK, d), lambda i: (core, i, 0))


def _full_spec(shape):
    return pl.BlockSpec(shape, lambda i: tuple(0 for _ in shape))


def _edges_body(e_ref, s_ref, d_ref):
    s_ref[...] = e_ref[0].reshape(_TOT_CHUNKS, _K)
    d_ref[...] = e_ref[1].reshape(_TOT_CHUNKS, _K)


def _edges(edge_index):
    return pl.pallas_call(
        _edges_body,
        out_shape=[jax.ShapeDtypeStruct((_TOT_CHUNKS, _K), jnp.int32)] * 2,
    )(edge_index)


def _tc1_body(x_ref, w_ref, xr_ref, xo_ref):
    cat = jnp.dot(x_ref[...], w_ref[...], preferred_element_type=jnp.float32)
    xr_ref[...] = cat[:, :64]
    xo_ref[...] = cat[:, 64:]


def _tc1(x, wcat):
    return pl.pallas_call(
        _tc1_body,
        grid=(_NBLK,),
        in_specs=[_row_spec(128), _full_spec((128, 128))],
        out_specs=[_row_spec(64), _row_spec(64)],
        out_shape=[jax.ShapeDtypeStruct((_N, 64), jnp.float32),
                   jax.ShapeDtypeStruct((_N, 64), jnp.float32)],
    )(x, wcat)


def _tc2_body(agg0_ref, agg1_ref, b1_ref, xo1_ref, w2_ref, h1_ref, xo2_ref):
    h1 = _elu(agg0_ref[0] + agg1_ref[0] + b1_ref[...] + xo1_ref[...])
    h1_ref[...] = h1
    xo2_ref[...] = jnp.dot(h1, w2_ref[...], preferred_element_type=jnp.float32)


def _tc2(agg, b1, xo1, wroot2t):
    return pl.pallas_call(
        _tc2_body,
        grid=(_NBLK,),
        in_specs=[_agg_spec(0, 64), _agg_spec(1, 64), _full_spec((1, 64)),
                  _row_spec(64), _full_spec((64, 128))],
        out_specs=[_row_spec(64), _row_spec(128)],
        out_shape=[jax.ShapeDtypeStruct((_N, 64), jnp.float32),
                   jax.ShapeDtypeStruct((_N, 128), jnp.float32)],
    )(agg, agg, b1, xo1, wroot2t)


def _tc3_body(agg0_ref, agg1_ref, b2_ref, xo2_ref, wrel2_ref, wroot3_ref,
              h2_ref, xo3_ref):
    a = agg0_ref[0] + agg1_ref[0]
    h2 = _elu(jnp.dot(a, wrel2_ref[...], preferred_element_type=jnp.float32)
              + b2_ref[...] + xo2_ref[...])
    h2_ref[...] = h2
    xo3_ref[...] = jnp.dot(h2, wroot3_ref[...],
                           preferred_element_type=jnp.float32)


def _tc3(agg, b2, xo2, wrel2t, wroot3t):
    return pl.pallas_call(
        _tc3_body,
        grid=(_NBLK,),
        in_specs=[_agg_spec(0, 64), _agg_spec(1, 64), _full_spec((1, 128)),
                  _row_spec(128), _full_spec((64, 128)),
                  _full_spec((128, 256))],
        out_specs=[_row_spec(128), _row_spec(256)],
        out_shape=[jax.ShapeDtypeStruct((_N, 128), jnp.float32),
                   jax.ShapeDtypeStruct((_N, 256), jnp.float32)],
    )(agg, agg, b2, xo2, wrel2t, wroot3t)


def _tc4_body(agg0_ref, agg1_ref, b3_ref, xo3_ref, wrel3_ref, batch_ref,
              wfc1_ref, bfc1_ref, wfc3_ref, bfc3_ref, o_ref,
              pooled_ref, cnt_ref):
    a = agg0_ref[0] + agg1_ref[0]
    h3 = _elu(jnp.dot(a, wrel3_ref[...], preferred_element_type=jnp.float32)
              + b3_ref[...] + xo3_ref[...])
    b = batch_ref[0, 0, :]
    gids = jax.lax.broadcasted_iota(jnp.int32, (128, _BLK), 0)
    maskt = (gids == b[None, :]).astype(jnp.float32)
    pooled_blk = jnp.dot(maskt, h3, preferred_element_type=jnp.float32)
    cnt_blk = jnp.sum(maskt, axis=1, keepdims=True)

    @pl.when(pl.program_id(0) == 0)
    def _():
        pooled_ref[...] = jnp.zeros_like(pooled_ref)
        cnt_ref[...] = jnp.zeros_like(cnt_ref)

    pooled_ref[...] += pooled_blk
    cnt_ref[...] += cnt_blk

    @pl.when(pl.program_id(0) == _NBLK - 1)
    def _():
        cnt = jnp.maximum(cnt_ref[...], 1.0)
        mean = pooled_ref[...] / cnt
        z = _elu(jnp.dot(mean, wfc1_ref[...],
                         preferred_element_type=jnp.float32)
                 + bfc1_ref[...])
        z2 = (jnp.dot(z, wfc3_ref[...], preferred_element_type=jnp.float32)
              + bfc3_ref[...])
        m = jnp.max(z2, axis=1, keepdims=True)
        ssum = jnp.sum(jnp.exp(z2 - m), axis=1, keepdims=True)
        o_ref[...] = z2 - m - jnp.log(ssum)


def _tc4(agg, b3, xo3, wrel3t, batch3d, wfc1t, bfc1, wfc3t_pad, bfc3_pad):
    return pl.pallas_call(
        _tc4_body,
        grid=(_NBLK,),
        in_specs=[_agg_spec(0, 128), _agg_spec(1, 128), _full_spec((1, 256)),
                  _row_spec(256), _full_spec((128, 256)),
                  pl.BlockSpec((1, 1, _BLK), lambda i: (i, 0, 0)),
                  _full_spec((256, 128)), _full_spec((1, 128)),
                  _full_spec((128, 128)), _full_spec((1, 128))],
        out_specs=_full_spec((128, 128)),
        out_shape=jax.ShapeDtypeStruct((128, 128), jnp.float32),
        scratch_shapes=[pltpu.VMEM((128, 256), jnp.float32),
                        pltpu.VMEM((128, 1), jnp.float32)],
    )(agg, agg, b3, xo3, wrel3t, batch3d, wfc1t, bfc1, wfc3t_pad, bfc3_pad)


def kernel(x, edge_index, batch, W_rel1, b_rel1, W_root1, W_rel2, b_rel2,
           W_root2, W_rel3, b_rel3, W_root3, W_fc1, b_fc1, W_fc3, b_fc3):
    src_p, dst_p = _edges(edge_index.astype(jnp.int32))

    zeros64 = jnp.zeros((_ROWS_PER_SUB, 64), jnp.float32)
    zeros128 = jnp.zeros((_ROWS_PER_SUB, 128), jnp.float32)

    wcat1 = jnp.concatenate([W_rel1.T, W_root1.T], axis=1)
    b1 = b_rel1.reshape(1, 64)
    b2 = b_rel2.reshape(1, 128)
    b3 = b_rel3.reshape(1, 256)
    batch3d = batch.astype(jnp.int32).reshape(_NBLK, 1, _BLK)

    wfc3t_pad = jnp.zeros((128, 128), jnp.float32).at[:, :_NUM_CLASSES].set(
        W_fc3.T)
    bfc3_pad = jnp.full((1, 128), -1e9, jnp.float32).at[0, :_NUM_CLASSES].set(
        b_fc3)

    xr1, xo1 = _tc1(x, wcat1)
    agg1 = _sc_segment_partials(xr1, src_p, dst_p, zeros64)
    h1, xo2 = _tc2(agg1, b1, xo1, W_root2.T)

    agg2 = _sc_segment_partials(h1, src_p, dst_p, zeros64)
    h2, xo3 = _tc3(agg2, b2, xo2, W_rel2.T, W_root3.T)

    agg3 = _sc_segment_partials(h2, src_p, dst_p, zeros128)
    out = _tc4(agg3, b3, xo3, W_rel3.T, batch3d, W_fc1.T,
               b_fc1.reshape(1, 128), wfc3t_pad, bfc3_pad)
    return out[:_NUM_GRAPHS, :_NUM_CLASSES]
